# Initial kernel scaffold; baseline (speedup 1.0000x reference)
import functools

import jax
import jax.numpy as jnp
from jax import lax
from jax.experimental import pallas as pl
from jax.experimental.pallas import tpu as pltpu
from jax.experimental.pallas import tpu_sc as plsc

E = 800000
CIN = 11
H = 64
B = 128
NC = 2
NS = 16
NW = NC * NS
CPW = 200
EP = NW * CPW * B
NCH = EP // B
BT = 1024


def _make_gather(C):
    mesh = plsc.VectorSubcoreMesh(core_axis_name="c", subcore_axis_name="s")

    @functools.partial(
        pl.kernel,
        out_type=jax.ShapeDtypeStruct((4, EP, C), jnp.float32),
        mesh=mesh,
        scratch_types=[
            pltpu.VMEM((2, 4, B), jnp.int32),
            pltpu.VMEM((2, 4, B, C), jnp.float32),
            pltpu.SemaphoreType.DMA,
            pltpu.SemaphoreType.DMA,
        ],
    )
    def gather_k(nb_hbm, h_hbm, out_hbm, idx_v, gath_v, gsem, wsem):
        wid = lax.axis_index("s") * NC + lax.axis_index("c")
        base = wid * CPW

        def body(g, carry):
            ch0 = base + 2 * g
            gcps = []
            for b in range(2):
                pltpu.sync_copy(nb_hbm.at[ch0 + b], idx_v.at[b])
                for j in range(4):
                    gcps.append(pltpu.async_copy(
                        h_hbm.at[idx_v.at[b, j]], gath_v.at[b, j], gsem))
            for cp in gcps:
                cp.wait()
            wcps = []
            for b in range(2):
                for j in range(4):
                    wcps.append(pltpu.async_copy(
                        gath_v.at[b, j],
                        out_hbm.at[j, pl.ds((ch0 + b) * B, B)], wsem))
            for cp in wcps:
                cp.wait()
            return carry

        lax.fori_loop(0, CPW // 2, body, None)

    return gather_k


_G16 = _make_gather(16)
_G64 = _make_gather(64)


def _conv(hb, g_ref, w_ref, cin):
    n0, n1, n2, n3 = g_ref[0], g_ref[1], g_ref[2], g_ref[3]
    feats = (hb, n0 + n2, jnp.abs(n0 - n2), n1 + n3, jnp.abs(n1 - n3))
    z = None
    for k, f in enumerate(feats):
        zk = jnp.dot(f, w_ref[k * cin:(k + 1) * cin, :],
                     preferred_element_type=jnp.float32)
        z = zk if z is None else z + zk
    return z


def _ln_relu(z, p_ref):
    z = z + p_ref[0][None, :]
    m = jnp.mean(z, axis=1, keepdims=True)
    zc = z - m
    v = jnp.mean(zc * zc, axis=1, keepdims=True)
    zn = zc * lax.rsqrt(v + 1e-5) * p_ref[1][None, :] + p_ref[2][None, :]
    return jnp.maximum(zn, 0.0)


def _tc0_body(h_ref, g_ref, w_ref, p_ref, o_ref):
    o_ref[...] = _ln_relu(_conv(h_ref[...], g_ref, w_ref, 16), p_ref)


def _tc_mid_body(h_ref, g_ref, w_ref, p_ref, o_ref):
    hb = h_ref[...]
    o_ref[...] = _ln_relu(_conv(hb, g_ref, w_ref, 64), p_ref) + hb


def _tc_last_body(h_ref, g_ref, w_ref, p_ref, cw1_ref, hp_ref, o_ref):
    hb = h_ref[...]
    a = _ln_relu(_conv(hb, g_ref, w_ref, 64), p_ref) + hb
    t = jnp.maximum(
        jnp.dot(a, cw1_ref[...], preferred_element_type=jnp.float32)
        + hp_ref[0, :32][None, :], 0.0)
    o_ref[...] = jnp.sum(t * hp_ref[1, :32][None, :], axis=1) + hp_ref[2, 0]


def _mk_tc(body, cin, out_shape, out_spec, extra_specs=()):
    return pl.pallas_call(
        body,
        grid=(EP // BT,),
        in_specs=[
            pl.BlockSpec((BT, cin), lambda i: (i, 0)),
            pl.BlockSpec((4, BT, cin), lambda i: (0, i, 0)),
            pl.BlockSpec((5 * cin, H), lambda i: (0, 0)),
            pl.BlockSpec((8, H), lambda i: (0, 0)),
            *extra_specs,
        ],
        out_specs=out_spec,
        out_shape=out_shape,
    )


_TC0 = _mk_tc(_tc0_body, 16,
              jax.ShapeDtypeStruct((EP, H), jnp.float32),
              pl.BlockSpec((BT, H), lambda i: (i, 0)))
_TCM = _mk_tc(_tc_mid_body, 64,
              jax.ShapeDtypeStruct((EP, H), jnp.float32),
              pl.BlockSpec((BT, H), lambda i: (i, 0)))
_TCL = _mk_tc(_tc_last_body, 64,
              jax.ShapeDtypeStruct((EP,), jnp.float32),
              pl.BlockSpec((BT,), lambda i: (i,)),
              extra_specs=(pl.BlockSpec((H, 32), lambda i: (0, 0)),
                           pl.BlockSpec((8, H), lambda i: (0, 0))))


def kernel(x, neighbors, W0, b0, g0, be0, W1, b1, g1, be1,
           W2, b2, g2, be2, W3, b3, g3, be3, cW1, cb1, cW2, cb2):
    xp = jnp.pad(x, ((0, EP - E), (0, 16 - CIN)))
    nbp = jnp.pad(neighbors, ((0, EP - E), (0, 0)))
    nb3 = nbp.T.reshape(4, NCH, B).transpose(1, 0, 2)

    w0p = jnp.zeros((80, H), jnp.float32)
    for k in range(5):
        w0p = w0p.at[k * 16:k * 16 + CIN].set(W0[k * CIN:(k + 1) * CIN])

    def pack(b, g, be):
        return jnp.concatenate(
            [b[None], g[None], be[None], jnp.zeros((5, H), jnp.float32)], 0)

    hp = jnp.zeros((8, H), jnp.float32)
    hp = hp.at[0, :32].set(cb1)
    hp = hp.at[1, :32].set(cW2[:, 0])
    hp = hp.at[2, 0].set(cb2[0])

    h = _TC0(xp, _G16(nb3, xp), w0p, pack(b0, g0, be0))
    h = _TCM(h, _G64(nb3, h), W1, pack(b1, g1, be1))
    h = _TCM(h, _G64(nb3, h), W2, pack(b2, g2, be2))
    out = _TCL(h, _G64(nb3, h), W3, pack(b3, g3, be3), cW1, hp)
    return out[:E]


# speedup vs baseline: 5.0676x; 5.0676x over previous
import functools

import jax
import jax.numpy as jnp
from jax import lax
from jax.experimental import pallas as pl
from jax.experimental.pallas import tpu as pltpu
from jax.experimental.pallas import tpu_sc as plsc

E = 800000
CIN = 11
H = 64
B = 128
NC = 2
NS = 16
NW = NC * NS
CPW = 200
EP = NW * CPW * B
NCH = EP // B
BT = 1024


@functools.lru_cache(maxsize=None)
def _make_gather(C):
    mesh = plsc.VectorSubcoreMesh(core_axis_name="c", subcore_axis_name="s")

    @functools.partial(
        pl.kernel,
        out_type=jax.ShapeDtypeStruct((4, EP, C), jnp.float32),
        mesh=mesh,
        compiler_params=pltpu.CompilerParams(use_tc_tiling_on_sc=False),
        scratch_types=[
            pltpu.VMEM((2, 4, B), jnp.int32),
            pltpu.VMEM((2, 4, B, C), jnp.float32),
            pltpu.SemaphoreType.DMA,
            pltpu.SemaphoreType.DMA,
        ],
    )
    def gather_k(nb_hbm, h_hbm, out_hbm, idx_v, gath_v, gsem, wsem):
        wid = lax.axis_index("s") * NC + lax.axis_index("c")
        base = wid * CPW

        def body(g, carry):
            ch0 = base + 2 * g
            gcps = []
            for b in range(2):
                pltpu.sync_copy(nb_hbm.at[ch0 + b], idx_v.at[b])
                for j in range(4):
                    gcps.append(pltpu.async_copy(
                        h_hbm.at[idx_v.at[b, j]], gath_v.at[b, j], gsem))
            for cp in gcps:
                cp.wait()
            wcps = []
            for b in range(2):
                for j in range(4):
                    wcps.append(pltpu.async_copy(
                        gath_v.at[b, j],
                        out_hbm.at[j, pl.ds((ch0 + b) * B, B)], wsem))
            for cp in wcps:
                cp.wait()
            return carry

        lax.fori_loop(0, CPW // 2, body, None)

    return gather_k


def _conv(hb, g_ref, w_ref, cin):
    n0, n1, n2, n3 = g_ref[0], g_ref[1], g_ref[2], g_ref[3]
    feats = (hb, n0 + n2, jnp.abs(n0 - n2), n1 + n3, jnp.abs(n1 - n3))
    z = None
    for k, f in enumerate(feats):
        zk = jnp.dot(f, w_ref[k * cin:(k + 1) * cin, :],
                     preferred_element_type=jnp.float32)
        z = zk if z is None else z + zk
    return z


def _ln_relu(z, p_ref):
    z = z + p_ref[0][None, :]
    m = jnp.mean(z, axis=1, keepdims=True)
    zc = z - m
    v = jnp.mean(zc * zc, axis=1, keepdims=True)
    zn = zc * lax.rsqrt(v + 1e-5) * p_ref[1][None, :] + p_ref[2][None, :]
    return jnp.maximum(zn, 0.0)


def _tc0_body(h_ref, g_ref, w_ref, p_ref, o_ref):
    o_ref[...] = _ln_relu(_conv(h_ref[...], g_ref, w_ref, 16), p_ref)


def _tc_mid_body(h_ref, g_ref, w_ref, p_ref, o_ref):
    hb = h_ref[...]
    o_ref[...] = _ln_relu(_conv(hb, g_ref, w_ref, 64), p_ref) + hb


def _tc_last_body(h_ref, g_ref, w_ref, p_ref, cw1_ref, hp_ref, o_ref):
    hb = h_ref[...]
    a = _ln_relu(_conv(hb, g_ref, w_ref, 64), p_ref) + hb
    t = jnp.maximum(
        jnp.dot(a, cw1_ref[...], preferred_element_type=jnp.float32)
        + hp_ref[0, :32][None, :], 0.0)
    o_ref[...] = jnp.sum(t * hp_ref[1, :32][None, :], axis=1) + hp_ref[2, 0]


def _mk_tc(body, cin, out_shape, out_spec, extra_specs=()):
    return pl.pallas_call(
        body,
        grid=(EP // BT,),
        in_specs=[
            pl.BlockSpec((BT, cin), lambda i: (i, 0)),
            pl.BlockSpec((4, BT, cin), lambda i: (0, i, 0)),
            pl.BlockSpec((5 * cin, H), lambda i: (0, 0)),
            pl.BlockSpec((8, H), lambda i: (0, 0)),
            *extra_specs,
        ],
        out_specs=out_spec,
        out_shape=out_shape,
    )


_TC0 = _mk_tc(_tc0_body, 16,
              jax.ShapeDtypeStruct((EP, H), jnp.float32),
              pl.BlockSpec((BT, H), lambda i: (i, 0)))
_TCM = _mk_tc(_tc_mid_body, 64,
              jax.ShapeDtypeStruct((EP, H), jnp.float32),
              pl.BlockSpec((BT, H), lambda i: (i, 0)))
_TCL = _mk_tc(_tc_last_body, 64,
              jax.ShapeDtypeStruct((EP,), jnp.float32),
              pl.BlockSpec((BT,), lambda i: (i,)),
              extra_specs=(pl.BlockSpec((H, 32), lambda i: (0, 0)),
                           pl.BlockSpec((8, H), lambda i: (0, 0))))


def kernel(x, neighbors, W0, b0, g0, be0, W1, b1, g1, be1,
           W2, b2, g2, be2, W3, b3, g3, be3, cW1, cb1, cW2, cb2):
    xp = jnp.pad(x, ((0, EP - E), (0, 16 - CIN)))
    nbp = jnp.pad(neighbors, ((0, EP - E), (0, 0)))
    nb3 = nbp.T.reshape(4, NCH, B).transpose(1, 0, 2)

    w0p = jnp.zeros((80, H), jnp.float32)
    for k in range(5):
        w0p = w0p.at[k * 16:k * 16 + CIN].set(W0[k * CIN:(k + 1) * CIN])

    def pack(b, g, be):
        return jnp.concatenate(
            [b[None], g[None], be[None], jnp.zeros((5, H), jnp.float32)], 0)

    hp = jnp.zeros((8, H), jnp.float32)
    hp = hp.at[0, :32].set(cb1)
    hp = hp.at[1, :32].set(cW2[:, 0])
    hp = hp.at[2, 0].set(cb2[0])

    g16, g64 = _make_gather(16), _make_gather(64)
    h = _TC0(xp, g16(nb3, xp), w0p, pack(b0, g0, be0))
    h = _TCM(h, g64(nb3, h), W1, pack(b1, g1, be1))
    h = _TCM(h, g64(nb3, h), W2, pack(b2, g2, be2))
    out = _TCL(h, g64(nb3, h), W3, pack(b3, g3, be3), cW1, hp)
    return out[:E]


# speedup vs baseline: 5.3066x; 1.0472x over previous
import functools

import jax
import jax.numpy as jnp
from jax import lax
from jax.experimental import pallas as pl
from jax.experimental.pallas import tpu as pltpu
from jax.experimental.pallas import tpu_sc as plsc

E = 800000
CIN = 11
H = 64
B = 128
NC = 2
NS = 16
NW = NC * NS
CPW = 200
EP = NW * CPW * B
NCH = EP // B
BT = 1024


SB = 20
NBUF = 3


@functools.lru_cache(maxsize=None)
def _make_gather(C):
    mesh = plsc.VectorSubcoreMesh(core_axis_name="c", subcore_axis_name="s")

    @functools.partial(
        pl.kernel,
        out_type=jax.ShapeDtypeStruct((4, EP, C), jnp.float32),
        mesh=mesh,
        compiler_params=pltpu.CompilerParams(use_tc_tiling_on_sc=False),
        scratch_types=[
            pltpu.VMEM((SB, 4, B), jnp.int32),
            pltpu.VMEM((NBUF, 4, B, C), jnp.float32),
        ] + [pltpu.SemaphoreType.DMA] * (2 * NBUF),
    )
    def gather_k(nb_hbm, h_hbm, out_hbm, idx_v, gath_v, *sems):
        gsems, wsems = sems[:NBUF], sems[NBUF:]
        wid = lax.axis_index("s") * NC + lax.axis_index("c")
        base = wid * CPW

        def super_body(t, carry):
            ch0 = base + t * SB
            pltpu.sync_copy(nb_hbm.at[pl.ds(ch0, SB)], idx_v)
            gcp, wcp = {}, {}

            def start_stores(kk):
                ss = kk % NBUF
                for c in gcp[kk]:
                    c.wait()
                wcp[kk] = [
                    pltpu.async_copy(gath_v.at[ss, j],
                                     out_hbm.at[j, pl.ds((ch0 + kk) * B, B)],
                                     wsems[ss])
                    for j in range(4)
                ]

            for k in range(SB):
                s = k % NBUF
                if k >= NBUF:
                    for c in wcp[k - NBUF]:
                        c.wait()
                gcp[k] = [
                    pltpu.async_copy(h_hbm.at[idx_v.at[k, j]],
                                     gath_v.at[s, j], gsems[s])
                    for j in range(4)
                ]
                if k >= 2:
                    start_stores(k - 2)
            start_stores(SB - 2)
            start_stores(SB - 1)
            for k in range(SB - NBUF, SB):
                for c in wcp[k]:
                    c.wait()
            return carry

        lax.fori_loop(0, CPW // SB, super_body, None)

    return gather_k


def _conv(hb, g_ref, w_ref, cin):
    n0, n1, n2, n3 = g_ref[0], g_ref[1], g_ref[2], g_ref[3]
    feats = (hb, n0 + n2, jnp.abs(n0 - n2), n1 + n3, jnp.abs(n1 - n3))
    z = None
    for k, f in enumerate(feats):
        zk = jnp.dot(f, w_ref[k * cin:(k + 1) * cin, :],
                     preferred_element_type=jnp.float32)
        z = zk if z is None else z + zk
    return z


def _ln_relu(z, p_ref):
    z = z + p_ref[0][None, :]
    m = jnp.mean(z, axis=1, keepdims=True)
    zc = z - m
    v = jnp.mean(zc * zc, axis=1, keepdims=True)
    zn = zc * lax.rsqrt(v + 1e-5) * p_ref[1][None, :] + p_ref[2][None, :]
    return jnp.maximum(zn, 0.0)


def _tc0_body(h_ref, g_ref, w_ref, p_ref, o_ref):
    o_ref[...] = _ln_relu(_conv(h_ref[...], g_ref, w_ref, 16), p_ref)


def _tc_mid_body(h_ref, g_ref, w_ref, p_ref, o_ref):
    hb = h_ref[...]
    o_ref[...] = _ln_relu(_conv(hb, g_ref, w_ref, 64), p_ref) + hb


def _tc_last_body(h_ref, g_ref, w_ref, p_ref, cw1_ref, hp_ref, o_ref):
    hb = h_ref[...]
    a = _ln_relu(_conv(hb, g_ref, w_ref, 64), p_ref) + hb
    t = jnp.maximum(
        jnp.dot(a, cw1_ref[...], preferred_element_type=jnp.float32)
        + hp_ref[0, :32][None, :], 0.0)
    o_ref[...] = jnp.sum(t * hp_ref[1, :32][None, :], axis=1) + hp_ref[2, 0]


def _mk_tc(body, cin, out_shape, out_spec, extra_specs=()):
    return pl.pallas_call(
        body,
        grid=(EP // BT,),
        in_specs=[
            pl.BlockSpec((BT, cin), lambda i: (i, 0)),
            pl.BlockSpec((4, BT, cin), lambda i: (0, i, 0)),
            pl.BlockSpec((5 * cin, H), lambda i: (0, 0)),
            pl.BlockSpec((8, H), lambda i: (0, 0)),
            *extra_specs,
        ],
        out_specs=out_spec,
        out_shape=out_shape,
    )


_TC0 = _mk_tc(_tc0_body, 16,
              jax.ShapeDtypeStruct((EP, H), jnp.float32),
              pl.BlockSpec((BT, H), lambda i: (i, 0)))
_TCM = _mk_tc(_tc_mid_body, 64,
              jax.ShapeDtypeStruct((EP, H), jnp.float32),
              pl.BlockSpec((BT, H), lambda i: (i, 0)))
_TCL = _mk_tc(_tc_last_body, 64,
              jax.ShapeDtypeStruct((EP,), jnp.float32),
              pl.BlockSpec((BT,), lambda i: (i,)),
              extra_specs=(pl.BlockSpec((H, 32), lambda i: (0, 0)),
                           pl.BlockSpec((8, H), lambda i: (0, 0))))


def kernel(x, neighbors, W0, b0, g0, be0, W1, b1, g1, be1,
           W2, b2, g2, be2, W3, b3, g3, be3, cW1, cb1, cW2, cb2):
    xp = jnp.pad(x, ((0, EP - E), (0, 16 - CIN)))
    nbp = jnp.pad(neighbors, ((0, EP - E), (0, 0)))
    nb3 = nbp.T.reshape(4, NCH, B).transpose(1, 0, 2)

    w0p = jnp.zeros((80, H), jnp.float32)
    for k in range(5):
        w0p = w0p.at[k * 16:k * 16 + CIN].set(W0[k * CIN:(k + 1) * CIN])

    def pack(b, g, be):
        return jnp.concatenate(
            [b[None], g[None], be[None], jnp.zeros((5, H), jnp.float32)], 0)

    hp = jnp.zeros((8, H), jnp.float32)
    hp = hp.at[0, :32].set(cb1)
    hp = hp.at[1, :32].set(cW2[:, 0])
    hp = hp.at[2, 0].set(cb2[0])

    g16, g64 = _make_gather(16), _make_gather(64)
    h = _TC0(xp, g16(nb3, xp), w0p, pack(b0, g0, be0))
    h = _TCM(h, g64(nb3, h), W1, pack(b1, g1, be1))
    h = _TCM(h, g64(nb3, h), W2, pack(b2, g2, be2))
    out = _TCL(h, g64(nb3, h), W3, pack(b3, g3, be3), cW1, hp)
    return out[:E]
